# baseline (speedup 1.0000x reference)
import functools

import jax
import jax.numpy as jnp
from jax import lax
from jax.experimental import pallas as pl
from jax.experimental.pallas import tpu as pltpu
from jax.experimental.pallas import tpu_sc as plsc

B = 16384
L = 200
D = 64
LH1 = 128
LH2 = L - LH1
NW = 32
BPW = B // NW
CH = 8
NBLK = BPW // CH
NBUF = 4
INV_L = 1.0 / L

_mesh = plsc.VectorSubcoreMesh(core_axis_name="c", subcore_axis_name="s")


@functools.partial(
    pl.kernel,
    mesh=_mesh,
    out_type=jax.ShapeDtypeStruct((B, D), jnp.float32),
    scratch_types=[
        pltpu.VMEM((2, CH, L), jnp.int32),
        pltpu.VMEM((NBUF, L, D // 2), jnp.float32),
        pltpu.VMEM((NBUF, L, D // 2), jnp.float32),
        pltpu.VMEM((CH, D), jnp.float32),
        [pltpu.SemaphoreType.DMA] * NBUF,
        [pltpu.SemaphoreType.DMA] * 2,
    ],
    compiler_params=pltpu.CompilerParams(use_tc_tiling_on_sc=False),
)
def _encode(x_hbm, tlo_hbm, thi_hbm, out_hbm, idx_v, rows_lo, rows_hi, out_v,
            gsem, isem):
    wid = lax.axis_index("s") * 2 + lax.axis_index("c")
    base = wid * BPW

    def fire_gather(q, j, p):
        for half, rows_v in ((tlo_hbm, rows_lo), (thi_hbm, rows_hi)):
            pltpu.async_copy(
                half.at[idx_v.at[q, j, pl.ds(0, LH1)]],
                rows_v.at[p, pl.ds(0, LH1)], gsem[p])
            pltpu.async_copy(
                half.at[idx_v.at[q, j, pl.ds(LH1, LH2)]],
                rows_v.at[p, pl.ds(LH1, LH2)], gsem[p])

    def wait_gather(q, j, p):
        for half, rows_v in ((tlo_hbm, rows_lo), (thi_hbm, rows_hi)):
            pltpu.make_async_copy(
                half.at[idx_v.at[q, j, pl.ds(0, LH1)]],
                rows_v.at[p, pl.ds(0, LH1)], gsem[p]).wait()
            pltpu.make_async_copy(
                half.at[idx_v.at[q, j, pl.ds(LH1, LH2)]],
                rows_v.at[p, pl.ds(LH1, LH2)], gsem[p]).wait()

    def reduce_rows(p, j):
        def red_body(i, acc):
            accs = list(acc)
            for rr in range(8):
                r = i * 8 + rr
                for c in range(2):
                    accs[c] = accs[c] + rows_lo[p, r, pl.ds(c * 16, 16)]
                    accs[2 + c] = accs[2 + c] + rows_hi[p, r, pl.ds(c * 16, 16)]
            return tuple(accs)

        zero = jnp.zeros((16,), jnp.float32)
        acc = lax.fori_loop(0, L // 8, red_body, (zero, zero, zero, zero))
        for c in range(4):
            out_v[j, pl.ds(c * 16, 16)] = acc[c] * INV_L

    def emit_block(blk, ip, fire_next, fire_idx):
        b0 = base + blk * CH
        for j in range(CH):
            p = j % NBUF
            if j < CH - 2:
                fire_gather(ip, j + 2, (j + 2) % NBUF)
            elif j == CH - 2:
                if fire_next:
                    pltpu.make_async_copy(
                        x_hbm.at[pl.ds(b0 + CH, CH)],
                        idx_v.at[1 - ip], isem[1 - ip]).wait()
                    fire_gather(1 - ip, 0, 0)
            else:
                if fire_next:
                    fire_gather(1 - ip, 1, 1)
                if fire_idx:
                    pltpu.async_copy(
                        x_hbm.at[pl.ds(b0 + 2 * CH, CH)],
                        idx_v.at[ip], isem[ip])
            wait_gather(ip, j, p)
            reduce_rows(p, j)
        pltpu.sync_copy(out_v, out_hbm.at[pl.ds(b0, CH)])

    pltpu.sync_copy(x_hbm.at[pl.ds(base, CH)], idx_v.at[0])
    pltpu.async_copy(x_hbm.at[pl.ds(base + CH, CH)], idx_v.at[1], isem[1])
    fire_gather(0, 0, 0)
    fire_gather(0, 1, 1)

    def pair_body(k, _):
        emit_block(2 * k, 0, True, True)
        emit_block(2 * k + 1, 1, True, True)
        return 0

    lax.fori_loop(0, NBLK // 2 - 1, pair_body, 0)
    emit_block(NBLK - 2, 0, True, False)
    emit_block(NBLK - 1, 1, False, False)


def kernel(x, table):
    return _encode(x, table[:, :D // 2], table[:, D // 2:])


# speedup vs baseline: 1.8976x; 1.8976x over previous
import functools

import jax
import jax.numpy as jnp
from jax import lax
from jax.experimental import pallas as pl
from jax.experimental.pallas import tpu as pltpu
from jax.experimental.pallas import tpu_sc as plsc

B = 16384
L = 200
D = 64
LH1 = 128
LH2 = L - LH1
NW = 32
BPW = B // NW
CH = 8
NBLK = BPW // CH
NBUF = 8
INV_L = 1.0 / L

_mesh = plsc.VectorSubcoreMesh(core_axis_name="c", subcore_axis_name="s")


@functools.partial(
    pl.kernel,
    mesh=_mesh,
    out_type=jax.ShapeDtypeStruct((B, D), jnp.float32),
    scratch_types=[
        pltpu.VMEM((2, CH, L), jnp.int32),
        pltpu.VMEM((NBUF, L, D), jnp.float32),
        pltpu.VMEM((CH, D), jnp.float32),
        [pltpu.SemaphoreType.DMA] * NBUF,
        [pltpu.SemaphoreType.DMA] * 2,
    ],
    compiler_params=pltpu.CompilerParams(use_tc_tiling_on_sc=False),
)
def _encode(x_hbm, table_hbm, out_hbm, idx_v, rows_v, out_v, gsem, isem):
    wid = lax.axis_index("s") * 2 + lax.axis_index("c")
    base = wid * BPW

    def fire_gather(q, j, p):
        pltpu.async_copy(
            table_hbm.at[idx_v.at[q, j, pl.ds(0, LH1)]],
            rows_v.at[p, pl.ds(0, LH1)], gsem[p])
        pltpu.async_copy(
            table_hbm.at[idx_v.at[q, j, pl.ds(LH1, LH2)]],
            rows_v.at[p, pl.ds(LH1, LH2)], gsem[p])

    def wait_gather(q, j, p):
        pltpu.make_async_copy(
            table_hbm.at[idx_v.at[q, j, pl.ds(0, LH1)]],
            rows_v.at[p, pl.ds(0, LH1)], gsem[p]).wait()
        pltpu.make_async_copy(
            table_hbm.at[idx_v.at[q, j, pl.ds(LH1, LH2)]],
            rows_v.at[p, pl.ds(LH1, LH2)], gsem[p]).wait()

    def reduce_rows(p, j):
        def red_body(i, acc):
            accs = list(acc)
            for rr in range(8):
                r = i * 8 + rr
                for c in range(4):
                    accs[c] = accs[c] + rows_v[p, r, pl.ds(c * 16, 16)]
            return tuple(accs)

        zero = jnp.zeros((16,), jnp.float32)
        acc = lax.fori_loop(0, L // 8, red_body, (zero, zero, zero, zero))
        for c in range(4):
            out_v[j, pl.ds(c * 16, 16)] = acc[c] * INV_L

    def emit_block(blk, ip, fire_next, fire_idx):
        b0 = base + blk * CH
        for j in range(CH):
            p = j % NBUF
            if j < CH - 4:
                fire_gather(ip, j + 4, (j + 4) % NBUF)
            elif j == CH - 4:
                if fire_next:
                    pltpu.make_async_copy(
                        x_hbm.at[pl.ds(b0 + CH, CH)],
                        idx_v.at[1 - ip], isem[1 - ip]).wait()
                    fire_gather(1 - ip, 0, 0)
            else:
                if fire_next:
                    fire_gather(1 - ip, j - 4, j - 4)
                if fire_idx and j == CH - 1:
                    pltpu.async_copy(
                        x_hbm.at[pl.ds(b0 + 2 * CH, CH)],
                        idx_v.at[ip], isem[ip])
            wait_gather(ip, j, p)
            reduce_rows(p, j)
        pltpu.sync_copy(out_v, out_hbm.at[pl.ds(b0, CH)])

    pltpu.sync_copy(x_hbm.at[pl.ds(base, CH)], idx_v.at[0])
    pltpu.async_copy(x_hbm.at[pl.ds(base + CH, CH)], idx_v.at[1], isem[1])
    for j in range(4):
        fire_gather(0, j, j)

    def pair_body(k, _):
        emit_block(2 * k, 0, True, True)
        emit_block(2 * k + 1, 1, True, True)
        return 0

    lax.fori_loop(0, NBLK // 2 - 1, pair_body, 0)
    emit_block(NBLK - 2, 0, True, False)
    emit_block(NBLK - 1, 1, False, False)


def kernel(x, table):
    return _encode(x, table)
